# baseline (speedup 1.0000x reference)
import functools

import jax
import jax.numpy as jnp
from jax import lax
from jax.experimental import pallas as pl
from jax.experimental.pallas import tpu as pltpu
from jax.experimental.pallas import tpu_sc as plsc

T, H, W, C = 2, 224, 224, 192
NUM_HEADS = 6
HD = C // NUM_HEADS
WS = 5
R = WS // 2
NPIX = T * H * W
NROW = T * H
RB = 8
LANES = 16


def _qkv_body(x_ref, wq_ref, wk_ref, wv_ref, bq_ref, bk_ref, bv_ref,
              qt_ref, kt_ref, vt_ref):
    scale = HD ** -0.5
    dn = (((0,), (1,)), ((), ()))
    wq, wk, wv = wq_ref[...], wk_ref[...], wv_ref[...]
    for rr in range(RB):
        x = x_ref[pl.ds(rr * W, W), :]
        q = lax.dot_general(wq, x, dn, preferred_element_type=jnp.float32)
        qt_ref[rr] = (q + bq_ref[...]) * scale
        k = lax.dot_general(wk, x, dn, preferred_element_type=jnp.float32)
        kt_ref[rr] = k + bk_ref[...]
        v = lax.dot_general(wv, x, dn, preferred_element_type=jnp.float32)
        vt_ref[rr] = v + bv_ref[...]


def _proj_body(ot_ref, wp_ref, bp_ref, out_ref):
    dn = (((0,), (0,)), ((), ()))
    for rr in range(RB):
        o = ot_ref[rr]
        y = lax.dot_general(o, wp_ref[...], dn,
                            preferred_element_type=jnp.float32)
        out_ref[pl.ds(rr * W, W), :] = y + bp_ref[...]


def _qkv_call(x, Wq, Wk, Wv, bq, bk, bv, interpret=False):
    nblk = NROW // RB
    wspec = pl.BlockSpec((C, C), lambda i: (0, 0))
    bspec = pl.BlockSpec((C, 1), lambda i: (0, 0))
    ospec = pl.BlockSpec((RB, C, W), lambda i: (i, 0, 0))
    oshape = jax.ShapeDtypeStruct((NROW, C, W), jnp.float32)
    return pl.pallas_call(
        _qkv_body,
        grid=(nblk,),
        in_specs=[pl.BlockSpec((RB * W, C), lambda i: (i, 0)),
                  wspec, wspec, wspec, bspec, bspec, bspec],
        out_specs=[ospec, ospec, ospec],
        out_shape=[oshape] * 3,
        interpret=interpret,
    )(x, Wq, Wk, Wv, bq, bk, bv)


def _proj_call(ot, Wp, bp, interpret=False):
    nblk = NROW // RB
    return pl.pallas_call(
        _proj_body,
        grid=(nblk,),
        in_specs=[pl.BlockSpec((RB, C, W), lambda i: (i, 0, 0)),
                  pl.BlockSpec((C, C), lambda i: (0, 0)),
                  pl.BlockSpec((1, C), lambda i: (0, 0))],
        out_specs=pl.BlockSpec((RB * W, C), lambda i: (i, 0)),
        out_shape=jax.ShapeDtypeStruct((NPIX, C), jnp.float32),
        interpret=interpret,
    )(ot, Wp, bp)


_SC_PARAMS = pltpu.CompilerParams(use_tc_tiling_on_sc=False,
                                  needs_layout_passes=False)


@functools.lru_cache(maxsize=None)
def _build_attn(t_, h_, w_, c_, heads):
    hd = c_ // heads
    nrow = t_ * h_
    nchunk = w_ // LANES
    nc, ns = 2, 16
    nworker = nc * ns
    rows_per_w = nrow // nworker
    ring = WS + 1
    mesh = plsc.VectorSubcoreMesh(core_axis_name="c", subcore_axis_name="s",
                                  num_cores=nc, num_subcores=ns)

    @functools.partial(
        pl.kernel,
        out_type=jax.ShapeDtypeStruct((nrow, c_, w_), jnp.float32),
        mesh=mesh,
        scratch_types=[
            pltpu.VMEM((ring, hd, w_), jnp.float32),
            pltpu.VMEM((ring, hd, w_), jnp.float32),
            pltpu.VMEM((hd, w_), jnp.float32),
            pltpu.VMEM((hd, w_), jnp.float32),
        ],
        compiler_params=_SC_PARAMS,
    )
    def attn(qt, kt, vt, ot, kbuf, vbuf, qbuf, obuf):
        cid = lax.axis_index("c")
        sid = lax.axis_index("s")
        wid = cid * ns + sid
        row0 = wid * rows_per_w
        t = row0 // h_
        y0 = row0 % h_
        rbase = t * h_

        def head_loop(n, _):
            ch0 = n * hd

            def pro(i, _):
                r = y0 - R + i

                @pl.when((r >= 0) & (r < h_))
                def _load():
                    slot = r % ring
                    pltpu.sync_copy(kt.at[rbase + r, pl.ds(ch0, hd)],
                                    kbuf.at[slot])
                    pltpu.sync_copy(vt.at[rbase + r, pl.ds(ch0, hd)],
                                    vbuf.at[slot])
                return 0

            lax.fori_loop(0, WS - 1, pro, 0)

            def row_loop(i, _):
                y = y0 + i
                rnew = y + R

                @pl.when(rnew < h_)
                def _load():
                    slot = rnew % ring
                    pltpu.sync_copy(kt.at[rbase + rnew, pl.ds(ch0, hd)],
                                    kbuf.at[slot])
                    pltpu.sync_copy(vt.at[rbase + rnew, pl.ds(ch0, hd)],
                                    vbuf.at[slot])

                pltpu.sync_copy(qt.at[rbase + y, pl.ds(ch0, hd)], qbuf)

                slots = []
                for o in range(-R, R + 1):
                    ry = jnp.abs(y + o)
                    ry = (h_ - 1) - jnp.abs((h_ - 1) - ry)
                    slots.append(ry % ring)

                def chunk_loop(xc, _):
                    x0 = xc * LANES
                    lane = lax.iota(jnp.int32, LANES) + x0
                    idxs = []
                    for o in range(-R, R + 1):
                        ix = jnp.abs(lane + o)
                        ix = (w_ - 1) - jnp.abs((w_ - 1) - ix)
                        idxs.append(ix)

                    def c_loop(cc, accs):
                        qv = qbuf[cc, pl.ds(x0, LANES)]
                        new = list(accs)
                        j = 0
                        for dy in range(WS):
                            for dx in range(WS):
                                kv = plsc.load_gather(kbuf.at[slots[dy], cc],
                                                      [idxs[dx]])
                                new[j] = new[j] + qv * kv
                                j += 1
                        return tuple(new)

                    zero = jnp.zeros((LANES,), jnp.float32)
                    accs = lax.fori_loop(0, hd, c_loop,
                                         tuple(zero for _ in range(WS * WS)))

                    m = accs[0]
                    for j in range(1, WS * WS):
                        m = jnp.maximum(m, accs[j])
                    es = [jnp.exp(a - m) for a in accs]
                    ssum = es[0]
                    for j in range(1, WS * WS):
                        ssum = ssum + es[j]
                    inv = 1.0 / ssum
                    wts = [e * inv for e in es]

                    def c2_loop(cc, _):
                        acc = zero
                        j = 0
                        for dy in range(WS):
                            for dx in range(WS):
                                vv = plsc.load_gather(vbuf.at[slots[dy], cc],
                                                      [idxs[dx]])
                                acc = acc + wts[j] * vv
                                j += 1
                        obuf[cc, pl.ds(x0, LANES)] = acc
                        return 0

                    lax.fori_loop(0, hd, c2_loop, 0)
                    return 0

                lax.fori_loop(0, nchunk, chunk_loop, 0)
                pltpu.sync_copy(obuf, ot.at[rbase + y, pl.ds(ch0, hd)])
                return 0

            lax.fori_loop(0, rows_per_w, row_loop, 0)
            return 0

        lax.fori_loop(0, heads, head_loop, 0)

    return attn


def kernel(vid, Wq, bq, Wk, bk, Wv, bv, Wp, bp):
    x = vid.reshape(NPIX, C)
    qt, kt, vt = _qkv_call(x, Wq, Wk, Wv, bq.reshape(C, 1), bk.reshape(C, 1),
                           bv.reshape(C, 1))
    attn = _build_attn(T, H, W, C, NUM_HEADS)
    ot = attn(qt, kt, vt)
    out = _proj_call(ot, Wp, bp.reshape(1, C))
    return out.reshape(T, H, W, C)


# speedup vs baseline: 1.0438x; 1.0438x over previous
import functools

import jax
import jax.numpy as jnp
from jax import lax
from jax.experimental import pallas as pl
from jax.experimental.pallas import tpu as pltpu
from jax.experimental.pallas import tpu_sc as plsc

T, H, W, C = 2, 224, 224, 192
NUM_HEADS = 6
HD = C // NUM_HEADS
WS = 5
R = WS // 2
NPIX = T * H * W
NROW = T * H
RB = 8
LANES = 16


def _qkv_body(x_ref, wq_ref, wk_ref, wv_ref, bq_ref, bk_ref, bv_ref,
              qt_ref, kt_ref, vt_ref):
    scale = HD ** -0.5
    dn = (((0,), (1,)), ((), ()))
    wq, wk, wv = wq_ref[...], wk_ref[...], wv_ref[...]
    for rr in range(RB):
        x = x_ref[pl.ds(rr * W, W), :]
        q = lax.dot_general(wq, x, dn, preferred_element_type=jnp.float32)
        qt_ref[rr] = (q + bq_ref[...]) * scale
        k = lax.dot_general(wk, x, dn, preferred_element_type=jnp.float32)
        kt_ref[rr] = k + bk_ref[...]
        v = lax.dot_general(wv, x, dn, preferred_element_type=jnp.float32)
        vt_ref[rr] = v + bv_ref[...]


def _proj_body(ot_ref, wp_ref, bp_ref, out_ref):
    dn = (((0,), (0,)), ((), ()))
    for rr in range(RB):
        o = ot_ref[rr]
        y = lax.dot_general(o, wp_ref[...], dn,
                            preferred_element_type=jnp.float32)
        out_ref[pl.ds(rr * W, W), :] = y + bp_ref[...]


def _qkv_call(x, Wq, Wk, Wv, bq, bk, bv, interpret=False):
    nblk = NROW // RB
    wspec = pl.BlockSpec((C, C), lambda i: (0, 0))
    bspec = pl.BlockSpec((C, 1), lambda i: (0, 0))
    ospec = pl.BlockSpec((RB, C, W), lambda i: (i, 0, 0))
    oshape = jax.ShapeDtypeStruct((NROW, C, W), jnp.float32)
    return pl.pallas_call(
        _qkv_body,
        grid=(nblk,),
        in_specs=[pl.BlockSpec((RB * W, C), lambda i: (i, 0)),
                  wspec, wspec, wspec, bspec, bspec, bspec],
        out_specs=[ospec, ospec, ospec],
        out_shape=[oshape] * 3,
        interpret=interpret,
    )(x, Wq, Wk, Wv, bq, bk, bv)


def _proj_call(ot, Wp, bp, interpret=False):
    nblk = NROW // RB
    return pl.pallas_call(
        _proj_body,
        grid=(nblk,),
        in_specs=[pl.BlockSpec((RB, C, W), lambda i: (i, 0, 0)),
                  pl.BlockSpec((C, C), lambda i: (0, 0)),
                  pl.BlockSpec((1, C), lambda i: (0, 0))],
        out_specs=pl.BlockSpec((RB * W, C), lambda i: (i, 0)),
        out_shape=jax.ShapeDtypeStruct((NPIX, C), jnp.float32),
        interpret=interpret,
    )(ot, Wp, bp)


_SC_PARAMS = pltpu.CompilerParams(use_tc_tiling_on_sc=False,
                                  needs_layout_passes=False)


@functools.lru_cache(maxsize=None)
def _build_attn(t_, h_, w_, c_, heads):
    hd = c_ // heads
    nrow = t_ * h_
    nchunk = w_ // LANES
    nc, ns = 2, 16
    nworker = nc * ns
    rows_per_w = nrow // nworker
    ring = WS + 1
    mesh = plsc.VectorSubcoreMesh(core_axis_name="c", subcore_axis_name="s",
                                  num_cores=nc, num_subcores=ns)

    cunroll = 2
    assert hd % cunroll == 0

    @functools.partial(
        pl.kernel,
        out_type=jax.ShapeDtypeStruct((nrow, c_, w_), jnp.float32),
        mesh=mesh,
        scratch_types=[
            pltpu.VMEM((ring, hd, w_), jnp.float32),
            pltpu.VMEM((ring, hd, w_), jnp.float32),
            pltpu.VMEM((2, hd, w_), jnp.float32),
            pltpu.VMEM((2, hd, w_), jnp.float32),
            pltpu.SemaphoreType.DMA,
            pltpu.SemaphoreType.DMA,
        ],
        compiler_params=_SC_PARAMS,
    )
    def attn(qt, kt, vt, ot, kbuf, vbuf, qbuf, obuf, sem_in, sem_out):
        cid = lax.axis_index("c")
        sid = lax.axis_index("s")
        wid = cid * ns + sid
        row0 = wid * rows_per_w
        t = row0 // h_
        y0 = row0 % h_
        rbase = t * h_

        def head_loop(n, _):
            ch0 = n * hd

            def pro(i, _):
                r = y0 - R + i

                @pl.when((r >= 0) & (r < h_))
                def _load():
                    slot = r % ring
                    pltpu.sync_copy(kt.at[rbase + r, pl.ds(ch0, hd)],
                                    kbuf.at[slot])
                    pltpu.sync_copy(vt.at[rbase + r, pl.ds(ch0, hd)],
                                    vbuf.at[slot])
                return 0

            lax.fori_loop(0, WS, pro, 0)
            pltpu.sync_copy(qt.at[rbase + y0, pl.ds(ch0, hd)], qbuf.at[0])

            def row_loop(i, _):
                y = y0 + i
                cur = i % 2
                nxt = (i + 1) % 2
                have_next = (i + 1) < rows_per_w
                rpre = y + R + 1
                pre_kv = have_next & (rpre < h_)

                @pl.when(have_next)
                def _pq():
                    pltpu.async_copy(qt.at[rbase + y + 1, pl.ds(ch0, hd)],
                                     qbuf.at[nxt], sem_in)

                @pl.when(pre_kv)
                def _pkv():
                    slot = rpre % ring
                    pltpu.async_copy(kt.at[rbase + rpre, pl.ds(ch0, hd)],
                                     kbuf.at[slot], sem_in)
                    pltpu.async_copy(vt.at[rbase + rpre, pl.ds(ch0, hd)],
                                     vbuf.at[slot], sem_in)

                slots = []
                for o in range(-R, R + 1):
                    ry = jnp.abs(y + o)
                    ry = (h_ - 1) - jnp.abs((h_ - 1) - ry)
                    slots.append(ry % ring)

                zero = jnp.zeros((LANES,), jnp.float32)

                def do_chunk(x0, edge):
                    if edge:
                        lane = lax.iota(jnp.int32, LANES) + x0
                        idxs = []
                        for o in range(-R, R + 1):
                            ix = jnp.abs(lane + o)
                            ix = (w_ - 1) - jnp.abs((w_ - 1) - ix)
                            idxs.append(ix)

                        def loadn(buf, dy, cc, dx):
                            return plsc.load_gather(buf.at[slots[dy], cc],
                                                    [idxs[dx]])
                    else:
                        def loadn(buf, dy, cc, dx):
                            return buf[slots[dy], cc,
                                       pl.ds(x0 + (dx - R), LANES)]

                    def c_loop(ci, accs):
                        new = list(accs)
                        for u in range(cunroll):
                            cc = ci * cunroll + u
                            qv = qbuf[cur, cc, pl.ds(x0, LANES)]
                            j = 0
                            for dy in range(WS):
                                for dx in range(WS):
                                    kv = loadn(kbuf, dy, cc, dx)
                                    new[j] = new[j] + qv * kv
                                    j += 1
                        return tuple(new)

                    accs = lax.fori_loop(0, hd // cunroll, c_loop,
                                         tuple(zero for _ in range(WS * WS)))

                    m = accs[0]
                    for j in range(1, WS * WS):
                        m = jnp.maximum(m, accs[j])
                    es = [jnp.exp(a - m) for a in accs]
                    ssum = es[0]
                    for j in range(1, WS * WS):
                        ssum = ssum + es[j]
                    inv = 1.0 / ssum
                    wts = [e * inv for e in es]

                    def c2_loop(ci, _):
                        for u in range(cunroll):
                            cc = ci * cunroll + u
                            acc = zero
                            j = 0
                            for dy in range(WS):
                                for dx in range(WS):
                                    vv = loadn(vbuf, dy, cc, dx)
                                    acc = acc + wts[j] * vv
                                    j += 1
                            obuf[cur, cc, pl.ds(x0, LANES)] = acc
                        return 0

                    lax.fori_loop(0, hd // cunroll, c2_loop, 0)

                do_chunk(0, True)

                def chunk_loop(xc, _):
                    do_chunk(xc * LANES, False)
                    return 0

                lax.fori_loop(1, nchunk - 1, chunk_loop, 0)
                do_chunk((nchunk - 1) * LANES, True)

                @pl.when(i > 0)
                def _wstore():
                    pltpu.make_async_copy(
                        obuf.at[nxt], ot.at[rbase + y - 1, pl.ds(ch0, hd)],
                        sem_out).wait()

                pltpu.async_copy(obuf.at[cur],
                                 ot.at[rbase + y, pl.ds(ch0, hd)], sem_out)

                @pl.when(have_next)
                def _wq():
                    pltpu.make_async_copy(
                        qt.at[rbase + y + 1, pl.ds(ch0, hd)], qbuf.at[nxt],
                        sem_in).wait()

                @pl.when(pre_kv)
                def _wkv():
                    slot = rpre % ring
                    pltpu.make_async_copy(
                        kt.at[rbase + rpre, pl.ds(ch0, hd)], kbuf.at[slot],
                        sem_in).wait()
                    pltpu.make_async_copy(
                        vt.at[rbase + rpre, pl.ds(ch0, hd)], vbuf.at[slot],
                        sem_in).wait()

                return 0

            lax.fori_loop(0, rows_per_w, row_loop, 0)
            pltpu.make_async_copy(
                obuf.at[(rows_per_w - 1) % 2],
                ot.at[rbase + y0 + rows_per_w - 1, pl.ds(ch0, hd)],
                sem_out).wait()
            return 0

        lax.fori_loop(0, heads, head_loop, 0)

    return attn


def kernel(vid, Wq, bq, Wk, bk, Wv, bv, Wp, bp):
    x = vid.reshape(NPIX, C)
    qt, kt, vt = _qkv_call(x, Wq, Wk, Wv, bq.reshape(C, 1), bk.reshape(C, 1),
                           bv.reshape(C, 1))
    attn = _build_attn(T, H, W, C, NUM_HEADS)
    ot = attn(qt, kt, vt)
    out = _proj_call(ot, Wp, bp.reshape(1, C))
    return out.reshape(T, H, W, C)


# speedup vs baseline: 2.1833x; 2.0917x over previous
import functools

import jax
import jax.numpy as jnp
from jax import lax
from jax.experimental import pallas as pl
from jax.experimental.pallas import tpu as pltpu
from jax.experimental.pallas import tpu_sc as plsc

T, H, W, C = 2, 224, 224, 192
NUM_HEADS = 6
HD = C // NUM_HEADS
WS = 5
R = WS // 2
NPIX = T * H * W
NROW = T * H
RB = 8
LANES = 16


def _qkv_body(x_ref, wq_ref, wk_ref, wv_ref, bq_ref, bk_ref, bv_ref,
              qt_ref, kt_ref, vt_ref):
    scale = HD ** -0.5
    dn = (((0,), (1,)), ((), ()))
    wq, wk, wv = wq_ref[...], wk_ref[...], wv_ref[...]
    for rr in range(RB):
        x = x_ref[pl.ds(rr * W, W), :]
        q = lax.dot_general(wq, x, dn, preferred_element_type=jnp.float32)
        qt_ref[rr] = (q + bq_ref[...]) * scale
        k = lax.dot_general(wk, x, dn, preferred_element_type=jnp.float32)
        kt_ref[rr] = k + bk_ref[...]
        v = lax.dot_general(wv, x, dn, preferred_element_type=jnp.float32)
        vt_ref[rr] = v + bv_ref[...]


def _proj_body(ot_ref, wp_ref, bp_ref, out_ref):
    dn = (((0,), (0,)), ((), ()))
    for rr in range(RB):
        o = ot_ref[rr]
        y = lax.dot_general(o, wp_ref[...], dn,
                            preferred_element_type=jnp.float32)
        out_ref[pl.ds(rr * W, W), :] = y + bp_ref[...]


def _qkv_call(x, Wq, Wk, Wv, bq, bk, bv, interpret=False):
    nblk = NROW // RB
    wspec = pl.BlockSpec((C, C), lambda i: (0, 0))
    bspec = pl.BlockSpec((C, 1), lambda i: (0, 0))
    ospec = pl.BlockSpec((RB, C, W), lambda i: (i, 0, 0))
    oshape = jax.ShapeDtypeStruct((NROW, C, W), jnp.float32)
    return pl.pallas_call(
        _qkv_body,
        grid=(nblk,),
        in_specs=[pl.BlockSpec((RB * W, C), lambda i: (i, 0)),
                  wspec, wspec, wspec, bspec, bspec, bspec],
        out_specs=[ospec, ospec, ospec],
        out_shape=[oshape] * 3,
        interpret=interpret,
    )(x, Wq, Wk, Wv, bq, bk, bv)


def _proj_call(ot, Wp, bp, interpret=False):
    nblk = NROW // RB
    return pl.pallas_call(
        _proj_body,
        grid=(nblk,),
        in_specs=[pl.BlockSpec((RB, C, W), lambda i: (i, 0, 0)),
                  pl.BlockSpec((C, C), lambda i: (0, 0)),
                  pl.BlockSpec((1, C), lambda i: (0, 0))],
        out_specs=pl.BlockSpec((RB * W, C), lambda i: (i, 0)),
        out_shape=jax.ShapeDtypeStruct((NPIX, C), jnp.float32),
        interpret=interpret,
    )(ot, Wp, bp)


_SC_PARAMS = pltpu.CompilerParams(use_tc_tiling_on_sc=False,
                                  needs_layout_passes=False)


@functools.lru_cache(maxsize=None)
def _build_attn(t_, h_, w_, c_, heads):
    hd = c_ // heads
    nrow = t_ * h_
    nchunk = w_ // LANES
    nc, ns = 2, 16
    nworker = nc * ns
    rows_per_w = nrow // nworker
    ring = WS + 1
    mesh = plsc.VectorSubcoreMesh(core_axis_name="c", subcore_axis_name="s",
                                  num_cores=nc, num_subcores=ns)

    cunroll = 2
    assert hd % cunroll == 0

    @functools.partial(
        pl.kernel,
        out_type=jax.ShapeDtypeStruct((nrow, c_, w_), jnp.float32),
        mesh=mesh,
        scratch_types=[
            pltpu.VMEM((ring, hd, w_), jnp.float32),
            pltpu.VMEM((ring, hd, w_), jnp.float32),
            pltpu.VMEM((2, hd, w_), jnp.float32),
            pltpu.VMEM((2, hd, w_), jnp.float32),
            pltpu.SemaphoreType.DMA,
            pltpu.SemaphoreType.DMA,
        ],
        compiler_params=_SC_PARAMS,
    )
    def attn(qt, kt, vt, ot, kbuf, vbuf, qbuf, obuf, sem_in, sem_out):
        cid = lax.axis_index("c")
        sid = lax.axis_index("s")
        wid = cid * ns + sid
        row0 = wid * rows_per_w
        t = row0 // h_
        y0 = row0 % h_
        rbase = t * h_

        def head_loop(n, _):
            ch0 = n * hd

            def pro(i, _):
                r = y0 - R + i

                @pl.when((r >= 0) & (r < h_))
                def _load():
                    slot = r % ring
                    pltpu.sync_copy(kt.at[rbase + r, pl.ds(ch0, hd)],
                                    kbuf.at[slot])
                    pltpu.sync_copy(vt.at[rbase + r, pl.ds(ch0, hd)],
                                    vbuf.at[slot])
                return 0

            lax.fori_loop(0, WS, pro, 0)
            pltpu.sync_copy(qt.at[rbase + y0, pl.ds(ch0, hd)], qbuf.at[0])

            def row_loop(i, _):
                y = y0 + i
                cur = i % 2
                nxt = (i + 1) % 2
                have_next = (i + 1) < rows_per_w
                rpre = y + R + 1
                pre_kv = have_next & (rpre < h_)

                @pl.when(have_next)
                def _pq():
                    pltpu.async_copy(qt.at[rbase + y + 1, pl.ds(ch0, hd)],
                                     qbuf.at[nxt], sem_in)

                @pl.when(pre_kv)
                def _pkv():
                    slot = rpre % ring
                    pltpu.async_copy(kt.at[rbase + rpre, pl.ds(ch0, hd)],
                                     kbuf.at[slot], sem_in)
                    pltpu.async_copy(vt.at[rbase + rpre, pl.ds(ch0, hd)],
                                     vbuf.at[slot], sem_in)

                slots = []
                for o in range(-R, R + 1):
                    ry = jnp.abs(y + o)
                    ry = (h_ - 1) - jnp.abs((h_ - 1) - ry)
                    slots.append(ry % ring)

                zero = jnp.zeros((LANES,), jnp.float32)

                def do_chunk(x0, edge):
                    if edge:
                        lane = lax.iota(jnp.int32, LANES) + x0
                        idxs = []
                        for o in range(-R, R + 1):
                            ix = jnp.abs(lane + o)
                            ix = (w_ - 1) - jnp.abs((w_ - 1) - ix)
                            idxs.append(ix)

                        def loadn(buf, dy, cc, dx):
                            return plsc.load_gather(buf.at[slots[dy], cc],
                                                    [idxs[dx]])
                    else:
                        def loadn(buf, dy, cc, dx):
                            return buf[slots[dy], cc,
                                       pl.ds(x0 + (dx - R), LANES)]

                    def c_loop(ci, accs):
                        new = list(accs)
                        for u in range(cunroll):
                            cc = ci * cunroll + u
                            qv = qbuf[cur, cc, pl.ds(x0, LANES)]
                            j = 0
                            for dy in range(WS):
                                for dx in range(WS):
                                    kv = loadn(kbuf, dy, cc, dx)
                                    new[j] = new[j] + qv * kv
                                    j += 1
                        return tuple(new)

                    accs = lax.fori_loop(0, hd // cunroll, c_loop,
                                         tuple(zero for _ in range(WS * WS)))

                    def _tree(vals, op):
                        vals = list(vals)
                        while len(vals) > 1:
                            nv = [op(vals[k], vals[k + 1])
                                  for k in range(0, len(vals) - 1, 2)]
                            if len(vals) % 2:
                                nv.append(vals[-1])
                            vals = nv
                        return vals[0]

                    m = _tree(accs, jnp.maximum)
                    es = [jnp.exp(a - m) for a in accs]
                    inv = 1.0 / _tree(es, jnp.add)
                    wts = [e * inv for e in es]

                    def c2_loop(ci, _):
                        for u in range(cunroll):
                            cc = ci * cunroll + u
                            parts = [zero] * WS
                            j = 0
                            for dy in range(WS):
                                for dx in range(WS):
                                    vv = loadn(vbuf, dy, cc, dx)
                                    parts[dy] = parts[dy] + wts[j] * vv
                                    j += 1
                            obuf[cur, cc, pl.ds(x0, LANES)] = \
                                _tree(parts, jnp.add)
                        return 0

                    lax.fori_loop(0, hd // cunroll, c2_loop, 0)

                do_chunk(0, True)

                @functools.partial(plsc.parallel_loop, 1, nchunk - 1)
                def chunk_loop(xc):
                    do_chunk(xc * LANES, False)

                do_chunk((nchunk - 1) * LANES, True)

                @pl.when(i > 0)
                def _wstore():
                    pltpu.make_async_copy(
                        obuf.at[nxt], ot.at[rbase + y - 1, pl.ds(ch0, hd)],
                        sem_out).wait()

                pltpu.async_copy(obuf.at[cur],
                                 ot.at[rbase + y, pl.ds(ch0, hd)], sem_out)

                @pl.when(have_next)
                def _wq():
                    pltpu.make_async_copy(
                        qt.at[rbase + y + 1, pl.ds(ch0, hd)], qbuf.at[nxt],
                        sem_in).wait()

                @pl.when(pre_kv)
                def _wkv():
                    slot = rpre % ring
                    pltpu.make_async_copy(
                        kt.at[rbase + rpre, pl.ds(ch0, hd)], kbuf.at[slot],
                        sem_in).wait()
                    pltpu.make_async_copy(
                        vt.at[rbase + rpre, pl.ds(ch0, hd)], vbuf.at[slot],
                        sem_in).wait()

                return 0

            lax.fori_loop(0, rows_per_w, row_loop, 0)
            pltpu.make_async_copy(
                obuf.at[(rows_per_w - 1) % 2],
                ot.at[rbase + y0 + rows_per_w - 1, pl.ds(ch0, hd)],
                sem_out).wait()
            return 0

        lax.fori_loop(0, heads, head_loop, 0)

    return attn


def kernel(vid, Wq, bq, Wk, bk, Wv, bv, Wp, bp):
    x = vid.reshape(NPIX, C)
    qt, kt, vt = _qkv_call(x, Wq, Wk, Wv, bq.reshape(C, 1), bk.reshape(C, 1),
                           bv.reshape(C, 1))
    attn = _build_attn(T, H, W, C, NUM_HEADS)
    ot = attn(qt, kt, vt)
    out = _proj_call(ot, Wp, bp.reshape(1, C))
    return out.reshape(T, H, W, C)
